# baseline (speedup 1.0000x reference)
import functools

import jax
import jax.numpy as jnp
from jax.experimental import pallas as pl

_N_E = 1024
_E_DIM = 256
_BETA = 0.25


def _vq_body(z_ref, cb_ref, zq_ref, idx_ref, loss_ref, *, scale):
    b = pl.program_id(0)
    C, H, W = z_ref.shape[1:]
    S = H * W
    zb = z_ref[0].reshape(C, S)
    cb = cb_ref[...]

    c2 = jnp.sum(cb * cb, axis=1, keepdims=True)
    z2 = jnp.sum(zb * zb, axis=0, keepdims=True)
    m = jax.lax.dot_general(
        cb, zb, (((1,), (0,)), ((), ())),
        preferred_element_type=jnp.float32)
    d = (z2 + c2) - 2.0 * m

    idx = jnp.argmin(d, axis=0).astype(jnp.int32)
    idx_ref[0, 0, :] = idx

    onehot = (jax.lax.broadcasted_iota(jnp.int32, (_N_E, S), 0)
              == idx[None, :]).astype(jnp.float32)
    zq = jax.lax.dot_general(
        cb, onehot, (((0,), (0,)), ((), ())),
        preferred_element_type=jnp.float32)
    zq_ref[0] = zq.reshape(C, H, W)

    diff = zq - zb
    part = (jnp.sum(diff * diff) * scale).reshape(1, 1)

    @pl.when(b == 0)
    def _():
        loss_ref[...] = jnp.zeros((1, 1), jnp.float32)

    loss_ref[...] += part


def kernel(z, codebook):
    B, C, H, W = z.shape
    S = H * W
    scale = (1.0 + _BETA) / (B * C * S)

    zq, idx3, loss = pl.pallas_call(
        functools.partial(_vq_body, scale=scale),
        grid=(B,),
        in_specs=[
            pl.BlockSpec((1, C, H, W), lambda b: (b, 0, 0, 0)),
            pl.BlockSpec((_N_E, _E_DIM), lambda b: (0, 0)),
        ],
        out_specs=[
            pl.BlockSpec((1, C, H, W), lambda b: (b, 0, 0, 0)),
            pl.BlockSpec((1, 1, S), lambda b: (b, 0, 0)),
            pl.BlockSpec((1, 1), lambda b: (0, 0)),
        ],
        out_shape=[
            jax.ShapeDtypeStruct((B, C, H, W), jnp.float32),
            jax.ShapeDtypeStruct((B, 1, S), jnp.int32),
            jax.ShapeDtypeStruct((1, 1), jnp.float32),
        ],
    )(z, codebook)

    indices_out = idx3.reshape(B, 1, H, W)
    return (zq, loss[0, 0], indices_out)


# speedup vs baseline: 1.0498x; 1.0498x over previous
import functools

import jax
import jax.numpy as jnp
from jax.experimental import pallas as pl
from jax.experimental.pallas import tpu as pltpu

_N_E = 1024
_E_DIM = 256
_BETA = 0.25


def _vq_body(z_hbm, cb_ref, zq_hbm, idx_ref, loss_ref,
             zin, zqv, in_sems, out_sems, *, scale, B, C, H, W):
    S = H * W
    b = pl.program_id(0)
    cur = jax.lax.rem(b, 2)
    nxt = jax.lax.rem(b + 1, 2)

    def in_copy(bi, slot, h):
        return pltpu.make_async_copy(
            z_hbm.at[bi, :, h, :],
            zin.at[slot, h],
            in_sems.at[slot, h])

    def out_copy(bi, slot, h):
        return pltpu.make_async_copy(
            zqv.at[slot, h],
            zq_hbm.at[bi, :, h, :],
            out_sems.at[slot, h])

    @pl.when(b == 0)
    def _():
        for h in range(H):
            in_copy(0, 0, h).start()

    @pl.when(b + 1 < B)
    def _():
        for h in range(H):
            in_copy(b + 1, nxt, h).start()

    for h in range(H):
        in_copy(b, cur, h).wait()

    zb = jnp.concatenate([zin[cur, h] for h in range(H)], axis=1)
    cb = cb_ref[...]

    c2 = jnp.sum(cb * cb, axis=1, keepdims=True)
    z2 = jnp.sum(zb * zb, axis=0, keepdims=True)
    m = jax.lax.dot_general(
        cb, zb, (((1,), (0,)), ((), ())),
        preferred_element_type=jnp.float32)
    d = (z2 + c2) - 2.0 * m

    idx = jnp.argmin(d, axis=0).astype(jnp.int32)
    idx_ref[0, 0, :] = idx

    onehot = (jax.lax.broadcasted_iota(jnp.int32, (_N_E, S), 0)
              == idx[None, :]).astype(jnp.float32)
    zq = jax.lax.dot_general(
        cb, onehot, (((0,), (0,)), ((), ())),
        preferred_element_type=jnp.float32)

    @pl.when(b >= 2)
    def _():
        for h in range(H):
            out_copy(b - 2, cur, h).wait()

    for h in range(H):
        zqv[cur, h] = zq[:, h * W:(h + 1) * W]

    diff = zq - zb
    part = (jnp.sum(diff * diff) * scale).reshape(1, 1)

    @pl.when(b == 0)
    def _():
        loss_ref[...] = jnp.zeros((1, 1), jnp.float32)

    loss_ref[...] += part

    for h in range(H):
        out_copy(b, cur, h).start()

    @pl.when(b == B - 1)
    def _():
        if B >= 2:
            for h in range(H):
                out_copy(b - 1, nxt, h).wait()
        for h in range(H):
            out_copy(b, cur, h).wait()


def kernel(z, codebook):
    B, C, H, W = z.shape
    S = H * W
    scale = (1.0 + _BETA) / (B * C * S)

    zq, idx3, loss = pl.pallas_call(
        functools.partial(_vq_body, scale=scale, B=B, C=C, H=H, W=W),
        grid=(B,),
        in_specs=[
            pl.BlockSpec(memory_space=pl.ANY),
            pl.BlockSpec((_N_E, _E_DIM), lambda b: (0, 0)),
        ],
        out_specs=[
            pl.BlockSpec(memory_space=pl.ANY),
            pl.BlockSpec((1, 1, S), lambda b: (b, 0, 0)),
            pl.BlockSpec((1, 1), lambda b: (0, 0)),
        ],
        out_shape=[
            jax.ShapeDtypeStruct((B, C, H, W), jnp.float32),
            jax.ShapeDtypeStruct((B, 1, S), jnp.int32),
            jax.ShapeDtypeStruct((1, 1), jnp.float32),
        ],
        scratch_shapes=[
            pltpu.VMEM((2, H, C, W), jnp.float32),
            pltpu.VMEM((2, H, C, W), jnp.float32),
            pltpu.SemaphoreType.DMA((2, H)),
            pltpu.SemaphoreType.DMA((2, H)),
        ],
    )(z, codebook)

    indices_out = idx3.reshape(B, 1, H, W)
    return (zq, loss[0, 0], indices_out)


# speedup vs baseline: 2.4595x; 2.3428x over previous
import functools

import jax
import jax.numpy as jnp
from jax.experimental import pallas as pl

_N_E = 1024
_E_DIM = 256
_BETA = 0.25


def _vq_body(z_ref, cb_ref, zq_ref, idx_ref, loss_ref, *, scale):
    b = pl.program_id(0)
    zb = z_ref[0]
    cb = cb_ref[...]
    S = zb.shape[1]
    cb16 = cb.astype(jnp.bfloat16)

    c2 = jnp.sum(cb * cb, axis=1, keepdims=True)
    z2 = jnp.sum(zb * zb, axis=0, keepdims=True)
    m = jax.lax.dot_general(
        cb16, zb.astype(jnp.bfloat16), (((1,), (0,)), ((), ())),
        preferred_element_type=jnp.float32)
    d = (z2 + c2) - 2.0 * m

    idx = jnp.argmin(d, axis=0).astype(jnp.int32)
    idx_ref[0, 0, :] = idx

    onehot = (jax.lax.broadcasted_iota(jnp.int32, (_N_E, S), 0)
              == idx[None, :]).astype(jnp.bfloat16)
    zq = jax.lax.dot_general(
        cb16, onehot, (((0,), (0,)), ((), ())),
        preferred_element_type=jnp.float32)
    zq_ref[0] = zq

    diff = zq - zb
    part = (jnp.sum(diff * diff) * scale).reshape(1, 1)

    @pl.when(b == 0)
    def _():
        loss_ref[...] = jnp.zeros((1, 1), jnp.float32)

    loss_ref[...] += part


def kernel(z, codebook):
    B, C, H, W = z.shape
    S = H * W
    z3 = z.reshape(B, C, S)
    scale = (1.0 + _BETA) / (B * C * S)

    zq3, idx3, loss = pl.pallas_call(
        functools.partial(_vq_body, scale=scale),
        grid=(B,),
        in_specs=[
            pl.BlockSpec((1, C, S), lambda b: (b, 0, 0)),
            pl.BlockSpec((_N_E, _E_DIM), lambda b: (0, 0)),
        ],
        out_specs=[
            pl.BlockSpec((1, C, S), lambda b: (b, 0, 0)),
            pl.BlockSpec((1, 1, S), lambda b: (b, 0, 0)),
            pl.BlockSpec((1, 1), lambda b: (0, 0)),
        ],
        out_shape=[
            jax.ShapeDtypeStruct((B, C, S), jnp.float32),
            jax.ShapeDtypeStruct((B, 1, S), jnp.int32),
            jax.ShapeDtypeStruct((1, 1), jnp.float32),
        ],
    )(z3, codebook)

    z_q_out = zq3.reshape(B, C, H, W)
    indices_out = idx3.reshape(B, 1, H, W)
    return (z_q_out, loss[0, 0], indices_out)
